# baseline (speedup 1.0000x reference)
import functools

import jax
import jax.numpy as jnp
from jax import lax
from jax.experimental import pallas as pl
from jax.experimental.pallas import tpu as pltpu
from jax.experimental.pallas import tpu_sc as plsc

N_AIDS = 1000000
EMBED_DIM = 64
BATCH = 16384

NC = 2
NS = 16
L = 16
NW = NC * NS
BPW = BATCH // NW
CH = 128
NCHUNK = BPW // CH


def _dot_kernel(aid1_hbm, aid2_hbm, emb_hbm, out_hbm,
                idx1_v, idx2_v, rows1_v, rows2_v, out_v, sem):
    wid = lax.axis_index("s") * NC + lax.axis_index("c")
    base = wid * BPW

    pltpu.sync_copy(aid1_hbm.at[pl.ds(base, BPW)], idx1_v)
    pltpu.sync_copy(aid2_hbm.at[pl.ds(base, BPW)], idx2_v)

    copies = []
    for c in range(NCHUNK):
        copies.append(pltpu.async_copy(
            emb_hbm.at[idx1_v.at[pl.ds(c * CH, CH)]],
            rows1_v.at[pl.ds(c * CH, CH)], sem))
        copies.append(pltpu.async_copy(
            emb_hbm.at[idx2_v.at[pl.ds(c * CH, CH)]],
            rows2_v.at[pl.ds(c * CH, CH)], sem))
    for cp in copies:
        cp.wait()

    lanes = lax.iota(jnp.int32, L)

    def block_body(c, carry):
        rb = c * L
        row_ids = rb + lanes
        acc = jnp.zeros((L,), jnp.float32)
        for d in range(EMBED_DIM):
            col = jnp.full((L,), d, jnp.int32)
            a = plsc.load_gather(rows1_v, [row_ids, col])
            b = plsc.load_gather(rows2_v, [row_ids, col])
            acc = acc + a * b
        out_v[pl.ds(rb, L)] = acc
        return carry

    lax.fori_loop(0, BPW // L, block_body, 0)

    pltpu.sync_copy(out_v, out_hbm.at[pl.ds(base, BPW)])


@jax.jit
def _run(aid1, aid2, embeddings):
    mesh = plsc.VectorSubcoreMesh(core_axis_name="c", subcore_axis_name="s")
    f = functools.partial(
        pl.kernel,
        mesh=mesh,
        compiler_params=pltpu.CompilerParams(
            needs_layout_passes=False, use_tc_tiling_on_sc=False),
        out_type=jax.ShapeDtypeStruct((BATCH,), jnp.float32),
        scratch_types=[
            pltpu.VMEM((BPW,), jnp.int32),
            pltpu.VMEM((BPW,), jnp.int32),
            pltpu.VMEM((BPW, EMBED_DIM), jnp.float32),
            pltpu.VMEM((BPW, EMBED_DIM), jnp.float32),
            pltpu.VMEM((BPW,), jnp.float32),
            pltpu.SemaphoreType.DMA,
        ],
    )(_dot_kernel)
    return f(aid1, aid2, embeddings)


def kernel(aid1, aid2, embeddings):
    return _run(aid1.astype(jnp.int32), aid2.astype(jnp.int32), embeddings)
